# Initial kernel scaffold; baseline (speedup 1.0000x reference)
import jax
import jax.numpy as jnp
from jax.experimental import pallas as pl


def kernel(x, exp, log, ands, ors):
    raise NotImplementedError("write your pallas kernel here")


# speedup vs baseline: 1.8642x; 1.8642x over previous
import functools

import jax
import jax.numpy as jnp
from jax import lax
from jax.experimental import pallas as pl
from jax.experimental.pallas import tpu as pltpu
from jax.experimental.pallas import tpu_sc as plsc

B = 512
N_ATOMS = 2048
N_ANDS = 4096
AND_SIZE = 8
N_ORS = 512
OR_SIZE = 8

NC = 2
NS = 16
NW = NC * NS
L = 16

ANDS_PER_W = N_ANDS // NW
ORS_PER_W = N_ORS // NW
G1 = 8
CH = B // L

_mesh = plsc.VectorSubcoreMesh(core_axis_name="c", subcore_axis_name="s")


@functools.partial(
    pl.kernel,
    out_type=jax.ShapeDtypeStruct((N_ANDS, B), jnp.float32),
    mesh=_mesh,
    scratch_types=[
        pltpu.VMEM((ANDS_PER_W * AND_SIZE,), jnp.int32),
        pltpu.VMEM((G1 * AND_SIZE, B), jnp.float32),
        pltpu.VMEM((G1, B), jnp.float32),
        pltpu.SemaphoreType.DMA,
    ],
)
def _stage1(xT_hbm, ands_hbm, out_hbm, idx_v, rows_v, prod_v, sem):
    wid = lax.axis_index("s") * NC + lax.axis_index("c")
    a0 = wid * ANDS_PER_W
    pltpu.sync_copy(ands_hbm.at[pl.ds(a0 * AND_SIZE, ANDS_PER_W * AND_SIZE)],
                    idx_v)

    def step(s, carry):
        idx_s = idx_v.at[pl.ds(s * (G1 * AND_SIZE), G1 * AND_SIZE)]
        pltpu.async_copy(xT_hbm.at[idx_s], rows_v, sem).wait()

        def chunk(c, carry2):
            off = pl.multiple_of(c * L, L)
            for g in range(G1):
                acc = rows_v[g * AND_SIZE, pl.ds(off, L)]
                for j in range(1, AND_SIZE):
                    acc = acc * rows_v[g * AND_SIZE + j, pl.ds(off, L)]
                prod_v[g, pl.ds(off, L)] = acc
            return carry2

        lax.fori_loop(0, CH, chunk, 0)
        pltpu.sync_copy(prod_v, out_hbm.at[pl.ds(a0 + s * G1, G1)])
        return carry

    lax.fori_loop(0, ANDS_PER_W // G1, step, 0)


@functools.partial(
    pl.kernel,
    out_type=jax.ShapeDtypeStruct((N_ORS, B), jnp.float32),
    mesh=_mesh,
    scratch_types=[
        pltpu.VMEM((ORS_PER_W * OR_SIZE,), jnp.int32),
        pltpu.VMEM((ORS_PER_W * OR_SIZE, B), jnp.float32),
        pltpu.VMEM((ORS_PER_W, B), jnp.float32),
        pltpu.SemaphoreType.DMA,
    ],
)
def _stage2(and_hbm, ors_hbm, out_hbm, idx_v, rows_v, acc_v, sem):
    wid = lax.axis_index("s") * NC + lax.axis_index("c")
    o0 = wid * ORS_PER_W
    pltpu.sync_copy(ors_hbm.at[pl.ds(o0 * OR_SIZE, ORS_PER_W * OR_SIZE)],
                    idx_v)
    pltpu.async_copy(and_hbm.at[idx_v], rows_v, sem).wait()

    def chunk(c, carry):
        off = pl.multiple_of(c * L, L)
        for g in range(ORS_PER_W):
            acc = 1.0 - rows_v[g * OR_SIZE, pl.ds(off, L)]
            for j in range(1, OR_SIZE):
                acc = acc * (1.0 - rows_v[g * OR_SIZE + j, pl.ds(off, L)])
            acc_v[g, pl.ds(off, L)] = 1.0 - acc
        return carry

    lax.fori_loop(0, CH, chunk, 0)
    pltpu.sync_copy(acc_v, out_hbm.at[pl.ds(o0, ORS_PER_W)])


def kernel(x, exp, log, ands, ors):
    del exp, log
    xT = jnp.transpose(jnp.reshape(x, (B, N_ATOMS)))
    and_emb_T = _stage1(xT, jnp.reshape(ands, (-1,)))
    out_T = _stage2(and_emb_T, jnp.reshape(ors, (-1,)))
    return jnp.transpose(out_T)


# speedup vs baseline: 1.8861x; 1.0118x over previous
import functools

import jax
import jax.numpy as jnp
from jax import lax
from jax.experimental import pallas as pl
from jax.experimental.pallas import tpu as pltpu
from jax.experimental.pallas import tpu_sc as plsc

B = 512
N_ATOMS = 2048
N_ANDS = 4096
AND_SIZE = 8
N_ORS = 512
OR_SIZE = 8

NC = 2
NS = 16
NW = NC * NS
L = 16

NCHUNK = 8
ANDS_PER_CHUNK = N_ANDS // NCHUNK
GPC = ANDS_PER_CHUNK // L
GOR = N_ORS // L

_mesh = plsc.VectorSubcoreMesh(core_axis_name="c", subcore_axis_name="s")


@functools.partial(
    pl.kernel,
    out_type=jax.ShapeDtypeStruct((B, N_ORS), jnp.float32),
    mesh=_mesh,
    compiler_params=pltpu.CompilerParams(needs_layout_passes=False),
    scratch_types=[
        pltpu.VMEM((L * N_ATOMS,), jnp.float32),
        pltpu.VMEM((L * N_ANDS,), jnp.float32),
        pltpu.VMEM((2, AND_SIZE, ANDS_PER_CHUNK), jnp.int32),
        pltpu.VMEM((OR_SIZE, N_ORS), jnp.int32),
        pltpu.VMEM((L, N_ORS), jnp.float32),
        pltpu.SemaphoreType.DMA,
        pltpu.SemaphoreType.DMA,
        pltpu.SemaphoreType.DMA,
    ],
)
def _reasoner(x_hbm, ands_hbm, ors_hbm, out_hbm,
              x_v, ae_v, andsc_v, ors_v, out_v, sem_x, sem_a, sem_b):
    wid = lax.axis_index("s") * NC + lax.axis_index("c")

    cp_x = pltpu.async_copy(
        x_hbm.at[pl.ds(wid * (L * N_ATOMS), L * N_ATOMS)], x_v, sem_x)
    cp_c = [None, None]
    cp_c[0] = pltpu.async_copy(ands_hbm.at[0], andsc_v.at[0], sem_a)
    cp_c[1] = pltpu.async_copy(ands_hbm.at[1], andsc_v.at[1], sem_b)
    pltpu.sync_copy(ors_hbm, ors_v)
    cp_x.wait()

    for c in range(NCHUNK):
        buf = c % 2
        cp_c[buf].wait()

        def group(g, carry, c=c, buf=buf):
            goff = pl.multiple_of(g * L, L)
            idx = [andsc_v[buf, j, pl.ds(goff, L)] for j in range(AND_SIZE)]
            for l in range(L):
                lo = l * N_ATOMS
                acc = plsc.load_gather(x_v, [idx[0] + lo])
                for j in range(1, AND_SIZE):
                    acc = acc * plsc.load_gather(x_v, [idx[j] + lo])
                ae_v[pl.ds(l * N_ANDS + c * ANDS_PER_CHUNK + goff, L)] = acc
            return carry

        lax.fori_loop(0, GPC, group, 0)
        if c + 2 < NCHUNK:
            cp_c[buf] = pltpu.async_copy(
                ands_hbm.at[c + 2], andsc_v.at[buf],
                sem_a if buf == 0 else sem_b)

    def group2(g, carry):
        goff = pl.multiple_of(g * L, L)
        idx = [ors_v[j, pl.ds(goff, L)] for j in range(OR_SIZE)]
        for l in range(L):
            lo = l * N_ANDS
            acc = 1.0 - plsc.load_gather(ae_v, [idx[0] + lo])
            for j in range(1, OR_SIZE):
                acc = acc * (1.0 - plsc.load_gather(ae_v, [idx[j] + lo]))
            out_v[l, pl.ds(goff, L)] = 1.0 - acc
        return carry

    lax.fori_loop(0, GOR, group2, 0)
    pltpu.sync_copy(out_v, out_hbm.at[pl.ds(wid * L, L)])


def kernel(x, exp, log, ands, ors):
    del exp, log
    xf = jnp.reshape(x, (-1,))
    ands_c = jnp.transpose(
        jnp.reshape(jnp.transpose(ands), (AND_SIZE, NCHUNK, ANDS_PER_CHUNK)),
        (1, 0, 2))
    orsT = jnp.transpose(ors)
    return _reasoner(xf, ands_c, orsT)


# speedup vs baseline: 2.0779x; 1.1016x over previous
import functools

import jax
import jax.numpy as jnp
from jax import lax
from jax.experimental import pallas as pl
from jax.experimental.pallas import tpu as pltpu
from jax.experimental.pallas import tpu_sc as plsc

B = 512
N_ATOMS = 2048
N_ANDS = 4096
AND_SIZE = 8
N_ORS = 512
OR_SIZE = 8

NC = 2
NS = 16
NW = NC * NS
L = 16

NCHUNK = 8
ANDS_PER_CHUNK = N_ANDS // NCHUNK
GPC = ANDS_PER_CHUNK // L
GOR = N_ORS // L

_mesh = plsc.VectorSubcoreMesh(core_axis_name="c", subcore_axis_name="s")


def _tree_prod(vals):
    while len(vals) > 1:
        nxt = [vals[i] * vals[i + 1] for i in range(0, len(vals) - 1, 2)]
        if len(vals) % 2:
            nxt.append(vals[-1])
        vals = nxt
    return vals[0]


@functools.partial(
    pl.kernel,
    out_type=jax.ShapeDtypeStruct((B, N_ORS), jnp.float32),
    mesh=_mesh,
    compiler_params=pltpu.CompilerParams(needs_layout_passes=False),
    scratch_types=[
        pltpu.VMEM((L * N_ATOMS,), jnp.float32),
        pltpu.VMEM((L * N_ANDS,), jnp.float32),
        pltpu.VMEM((2, AND_SIZE, ANDS_PER_CHUNK), jnp.int32),
        pltpu.VMEM((OR_SIZE, N_ORS), jnp.int32),
        pltpu.VMEM((L, N_ORS), jnp.float32),
        pltpu.SemaphoreType.DMA,
        pltpu.SemaphoreType.DMA,
        pltpu.SemaphoreType.DMA,
    ],
)
def _reasoner(x_hbm, ands_hbm, ors_hbm, out_hbm,
              x_v, ae_v, andsc_v, ors_v, out_v, sem_x, sem_a, sem_b):
    wid = lax.axis_index("s") * NC + lax.axis_index("c")

    cp_x = pltpu.async_copy(
        x_hbm.at[pl.ds(wid * (L * N_ATOMS), L * N_ATOMS)], x_v, sem_x)
    cp_c = [pltpu.async_copy(ands_hbm.at[0], andsc_v.at[0], sem_a),
            pltpu.async_copy(ands_hbm.at[1], andsc_v.at[1], sem_b)]
    pltpu.sync_copy(ors_hbm, ors_v)
    cp_x.wait()

    for c in range(NCHUNK):
        buf = c % 2
        cp_c[buf].wait()

        def make_group(c, buf):
            def group(g):
                goff = pl.multiple_of(g * L, L)
                idx = [andsc_v[buf, j, pl.ds(goff, L)]
                       for j in range(AND_SIZE)]
                for l in range(L):
                    lo = l * N_ATOMS
                    acc = _tree_prod(
                        [plsc.load_gather(x_v, [idx[j] + lo])
                         for j in range(AND_SIZE)])
                    ae_v[pl.ds(l * N_ANDS + c * ANDS_PER_CHUNK + goff, L)] = acc
            return group

        def wrap(g, carry, c=c, buf=buf):
            make_group(c, buf)(g)
            return carry
        lax.fori_loop(0, GPC, wrap, 0)
        if c + 2 < NCHUNK:
            cp_c[buf] = pltpu.async_copy(
                ands_hbm.at[c + 2], andsc_v.at[buf],
                sem_a if buf == 0 else sem_b)

    def group2(g, carry):
        goff = pl.multiple_of(g * L, L)
        idx = [ors_v[j, pl.ds(goff, L)] for j in range(OR_SIZE)]
        for l in range(L):
            lo = l * N_ANDS
            acc = _tree_prod(
                [1.0 - plsc.load_gather(ae_v, [idx[j] + lo])
                 for j in range(OR_SIZE)])
            out_v[l, pl.ds(goff, L)] = 1.0 - acc
        return carry

    lax.fori_loop(0, GOR, group2, 0)
    pltpu.sync_copy(out_v, out_hbm.at[pl.ds(wid * L, L)])


def kernel(x, exp, log, ands, ors):
    del exp, log
    xf = jnp.reshape(x, (-1,))
    ands_c = jnp.transpose(
        jnp.reshape(jnp.transpose(ands), (AND_SIZE, NCHUNK, ANDS_PER_CHUNK)),
        (1, 0, 2))
    orsT = jnp.transpose(ors)
    return _reasoner(xf, ands_c, orsT)
